# Initial kernel scaffold; baseline (speedup 1.0000x reference)
import jax
import jax.numpy as jnp
from jax.experimental import pallas as pl


def kernel(uvs, texture):
    raise NotImplementedError("write your pallas kernel here")


# speedup vs baseline: 3.9267x; 3.9267x over previous
import functools

import jax
import jax.numpy as jnp
from jax import lax
from jax.experimental import pallas as pl
from jax.experimental.pallas import tpu as pltpu
from jax.experimental.pallas import tpu_sc as plsc

L = 16
IDX_MINOR = 128
CHUNK = 2048
KGATH = CHUNK // IDX_MINOR
NW = 32


def _nearest(x, scale):
    x01 = (x + 1.0) * 0.5
    xf = x01 * scale
    x0 = xf.astype(jnp.int32)
    frac = xf - x0.astype(jnp.float32)
    xi = jnp.where(frac >= 0.5, x0 + 1, x0)
    return jnp.minimum(xi, 4095)


def _tex_kernel(n_queries, h, w, us_hbm, vs_hbm, tex_hbm,
                r_hbm, g_hbm, b_hbm,
                u_v, v_v, idx_v, cr_v, cg_v, cb_v, sem):
    nc = 2
    wid = lax.axis_index("s") * nc + lax.axis_index("c")
    per_w = n_queries // NW
    n_chunks = per_w // CHUNK
    plane = h * w

    def chunk_body(g, carry):
        base = wid * per_w + g * CHUNK
        pltpu.sync_copy(us_hbm.at[pl.ds(base, CHUNK)], u_v)
        pltpu.sync_copy(vs_hbm.at[pl.ds(base, CHUNK)], v_v)

        def idx_body(j, c):
            for l in range(IDX_MINOR // L):
                off = j * IDX_MINOR + l * L
                u = u_v[pl.ds(off, L)]
                v = v_v[pl.ds(off, L)]
                iu = _nearest(u, float(w - 1))
                iv = _nearest(v, float(h - 1))
                addr = (((iu >> 3) << 15) | ((iv >> 7) << 10)
                        | ((iu & 7) << 7) | (iv & 127))
                idx_v[j, pl.ds(l * L, L)] = addr
                idx_v[j + KGATH, pl.ds(l * L, L)] = addr + plane
                idx_v[j + 2 * KGATH, pl.ds(l * L, L)] = addr + 2 * plane
            return c

        lax.fori_loop(0, KGATH, idx_body, 0)

        handles = []
        for j in range(KGATH):
            dst = pl.ds(j * IDX_MINOR, IDX_MINOR)
            handles.append(pltpu.async_copy(
                tex_hbm.at[idx_v.at[j]], cr_v.at[dst], sem))
            handles.append(pltpu.async_copy(
                tex_hbm.at[idx_v.at[j + KGATH]], cg_v.at[dst], sem))
            handles.append(pltpu.async_copy(
                tex_hbm.at[idx_v.at[j + 2 * KGATH]], cb_v.at[dst], sem))
        for hdl in handles:
            hdl.wait()

        pltpu.sync_copy(cr_v, r_hbm.at[pl.ds(base, CHUNK)])
        pltpu.sync_copy(cg_v, g_hbm.at[pl.ds(base, CHUNK)])
        pltpu.sync_copy(cb_v, b_hbm.at[pl.ds(base, CHUNK)])
        return carry

    lax.fori_loop(0, n_chunks, chunk_body, 0)


def kernel(uvs, texture):
    n = uvs.shape[0]
    h, w, _ = texture.shape
    uvs_t = uvs.T
    us = uvs_t[0]
    vs = uvs_t[1]
    tex_lin = (texture.transpose(2, 0, 1)
               .reshape(3, h // 8, 8, w // 128, 128)
               .transpose(0, 1, 3, 2, 4)
               .reshape(3 * h * w))

    mesh = plsc.VectorSubcoreMesh(core_axis_name="c", subcore_axis_name="s")
    out1d = jax.ShapeDtypeStruct((n,), jnp.float32)
    run = functools.partial(
        pl.kernel,
        mesh=mesh,
        out_type=(out1d, out1d, out1d),
        scratch_types=[
            pltpu.VMEM((CHUNK,), jnp.float32),
            pltpu.VMEM((CHUNK,), jnp.float32),
            pltpu.VMEM((3 * KGATH, IDX_MINOR), jnp.int32),
            pltpu.VMEM((CHUNK,), jnp.float32),
            pltpu.VMEM((CHUNK,), jnp.float32),
            pltpu.VMEM((CHUNK,), jnp.float32),
            pltpu.SemaphoreType.DMA,
        ],
        compiler_params=pltpu.CompilerParams(use_tc_tiling_on_sc=False),
    )(functools.partial(_tex_kernel, n, h, w))
    r, g, b = run(us, vs, tex_lin)
    return jnp.stack([r, g, b], axis=1)


# speedup vs baseline: 3.9365x; 1.0025x over previous
import functools

import jax
import jax.numpy as jnp
from jax import lax
from jax.experimental import pallas as pl
from jax.experimental.pallas import tpu as pltpu
from jax.experimental.pallas import tpu_sc as plsc

L = 16
IDX_MINOR = 128
CHUNK = 2048
KGATH = CHUNK // IDX_MINOR
NW = 32


def _nearest(x, scale):
    x01 = (x + 1.0) * 0.5
    xf = x01 * scale
    x0 = xf.astype(jnp.int32)
    frac = xf - x0.astype(jnp.float32)
    xi = jnp.where(frac >= 0.5, x0 + 1, x0)
    return jnp.minimum(xi, 4095)


def _tex_kernel(n_queries, h, w, us_hbm, vs_hbm, tex_hbm,
                r_hbm, g_hbm, b_hbm,
                u_v, v_v, idx_v, cr_v, cg_v, cb_v, sem):
    nc = 2
    wid = lax.axis_index("s") * nc + lax.axis_index("c")
    per_w = n_queries // NW
    n_chunks = per_w // CHUNK
    plane = h * w

    def chunk_body(g, carry):
        base = wid * per_w + g * CHUNK
        pltpu.sync_copy(us_hbm.at[pl.ds(base, CHUNK)], u_v)
        pltpu.sync_copy(vs_hbm.at[pl.ds(base, CHUNK)], v_v)

        def idx_body(j, c):
            for l in range(IDX_MINOR // L):
                off = j * IDX_MINOR + l * L
                u = u_v[pl.ds(off, L)]
                v = v_v[pl.ds(off, L)]
                iu = _nearest(u, float(w - 1))
                iv = _nearest(v, float(h - 1))
                addr = (((iu >> 3) << 15) | ((iv >> 7) << 10)
                        | ((iu & 7) << 7) | (iv & 127))
                idx_v[0, pl.ds(off, L)] = addr
                idx_v[1, pl.ds(off, L)] = addr + plane
                idx_v[2, pl.ds(off, L)] = addr + 2 * plane
            return c

        lax.fori_loop(0, KGATH, idx_body, 0)

        handles = [
            pltpu.async_copy(tex_hbm.at[idx_v.at[0]], cr_v, sem),
            pltpu.async_copy(tex_hbm.at[idx_v.at[1]], cg_v, sem),
            pltpu.async_copy(tex_hbm.at[idx_v.at[2]], cb_v, sem),
        ]
        for hdl in handles:
            hdl.wait()

        pltpu.sync_copy(cr_v, r_hbm.at[pl.ds(base, CHUNK)])
        pltpu.sync_copy(cg_v, g_hbm.at[pl.ds(base, CHUNK)])
        pltpu.sync_copy(cb_v, b_hbm.at[pl.ds(base, CHUNK)])
        return carry

    lax.fori_loop(0, n_chunks, chunk_body, 0)


def kernel(uvs, texture):
    n = uvs.shape[0]
    h, w, _ = texture.shape
    uvs_t = uvs.T
    us = uvs_t[0]
    vs = uvs_t[1]
    tex_lin = (texture.transpose(2, 0, 1)
               .reshape(3, h // 8, 8, w // 128, 128)
               .transpose(0, 1, 3, 2, 4)
               .reshape(3 * h * w))

    mesh = plsc.VectorSubcoreMesh(core_axis_name="c", subcore_axis_name="s")
    out1d = jax.ShapeDtypeStruct((n,), jnp.float32)
    run = functools.partial(
        pl.kernel,
        mesh=mesh,
        out_type=(out1d, out1d, out1d),
        scratch_types=[
            pltpu.VMEM((CHUNK,), jnp.float32),
            pltpu.VMEM((CHUNK,), jnp.float32),
            pltpu.VMEM((3, CHUNK), jnp.int32),
            pltpu.VMEM((CHUNK,), jnp.float32),
            pltpu.VMEM((CHUNK,), jnp.float32),
            pltpu.VMEM((CHUNK,), jnp.float32),
            pltpu.SemaphoreType.DMA,
        ],
        compiler_params=pltpu.CompilerParams(use_tc_tiling_on_sc=False),
    )(functools.partial(_tex_kernel, n, h, w))
    r, g, b = run(us, vs, tex_lin)
    return jnp.stack([r, g, b], axis=1)


# speedup vs baseline: 4.5196x; 1.1481x over previous
import functools

import jax
import jax.numpy as jnp
from jax import lax
from jax.experimental import pallas as pl
from jax.experimental.pallas import tpu as pltpu
from jax.experimental.pallas import tpu_sc as plsc

L = 16
IDX_MINOR = 128
CHUNK = 8192
KGATH = CHUNK // IDX_MINOR
NW = 32


def _nearest(x, scale):
    x01 = (x + 1.0) * 0.5
    xf = x01 * scale
    x0 = xf.astype(jnp.int32)
    frac = xf - x0.astype(jnp.float32)
    xi = jnp.where(frac >= 0.5, x0 + 1, x0)
    return jnp.minimum(xi, 4095)


def _tex_kernel(n_queries, h, w, us_hbm, vs_hbm, tex_hbm,
                r_hbm, g_hbm, b_hbm,
                u_v, v_v, idx_v, cr_v, cg_v, cb_v, sem):
    nc = 2
    wid = lax.axis_index("s") * nc + lax.axis_index("c")
    per_w = n_queries // NW
    n_chunks = per_w // CHUNK
    plane = h * w

    def chunk_body(g, carry):
        base = wid * per_w + g * CHUNK
        pltpu.sync_copy(us_hbm.at[pl.ds(base, CHUNK)], u_v)
        pltpu.sync_copy(vs_hbm.at[pl.ds(base, CHUNK)], v_v)

        def idx_body(j, c):
            for l in range(IDX_MINOR // L):
                off = j * IDX_MINOR + l * L
                u = u_v[pl.ds(off, L)]
                v = v_v[pl.ds(off, L)]
                iu = _nearest(u, float(w - 1))
                iv = _nearest(v, float(h - 1))
                addr = (((iu >> 3) << 15) | ((iv >> 7) << 10)
                        | ((iu & 7) << 7) | (iv & 127))
                idx_v[0, pl.ds(off, L)] = addr
                idx_v[1, pl.ds(off, L)] = addr + plane
                idx_v[2, pl.ds(off, L)] = addr + 2 * plane
            return c

        lax.fori_loop(0, KGATH, idx_body, 0)

        handles = [
            pltpu.async_copy(tex_hbm.at[idx_v.at[0]], cr_v, sem),
            pltpu.async_copy(tex_hbm.at[idx_v.at[1]], cg_v, sem),
            pltpu.async_copy(tex_hbm.at[idx_v.at[2]], cb_v, sem),
        ]
        for hdl in handles:
            hdl.wait()

        pltpu.sync_copy(cr_v, r_hbm.at[pl.ds(base, CHUNK)])
        pltpu.sync_copy(cg_v, g_hbm.at[pl.ds(base, CHUNK)])
        pltpu.sync_copy(cb_v, b_hbm.at[pl.ds(base, CHUNK)])
        return carry

    lax.fori_loop(0, n_chunks, chunk_body, 0)


def kernel(uvs, texture):
    n = uvs.shape[0]
    h, w, _ = texture.shape
    uvs_t = uvs.T
    us = uvs_t[0]
    vs = uvs_t[1]
    tex_lin = (texture.transpose(2, 0, 1)
               .reshape(3, h // 8, 8, w // 128, 128)
               .transpose(0, 1, 3, 2, 4)
               .reshape(3 * h * w))

    mesh = plsc.VectorSubcoreMesh(core_axis_name="c", subcore_axis_name="s")
    out1d = jax.ShapeDtypeStruct((n,), jnp.float32)
    run = functools.partial(
        pl.kernel,
        mesh=mesh,
        out_type=(out1d, out1d, out1d),
        scratch_types=[
            pltpu.VMEM((CHUNK,), jnp.float32),
            pltpu.VMEM((CHUNK,), jnp.float32),
            pltpu.VMEM((3, CHUNK), jnp.int32),
            pltpu.VMEM((CHUNK,), jnp.float32),
            pltpu.VMEM((CHUNK,), jnp.float32),
            pltpu.VMEM((CHUNK,), jnp.float32),
            pltpu.SemaphoreType.DMA,
        ],
        compiler_params=pltpu.CompilerParams(use_tc_tiling_on_sc=False),
    )(functools.partial(_tex_kernel, n, h, w))
    r, g, b = run(us, vs, tex_lin)
    return jnp.stack([r, g, b], axis=1)


# speedup vs baseline: 5.3412x; 1.1818x over previous
import functools

import jax
import jax.numpy as jnp
from jax import lax
from jax.experimental import pallas as pl
from jax.experimental.pallas import tpu as pltpu
from jax.experimental.pallas import tpu_sc as plsc

L = 16
BLK = 128
CHUNK = 4096
NW = 32


def _nearest(x, scale):
    x01 = (x + 1.0) * 0.5
    xf = x01 * scale
    x0 = xf.astype(jnp.int32)
    frac = xf - x0.astype(jnp.float32)
    xi = jnp.where(frac >= 0.5, x0 + 1, x0)
    return jnp.minimum(xi, 4095)


def _tex_kernel(n_queries, h, w, uv_hbm, tex_hbm, out_hbm,
                uv0, uv1, idx0, idx1, ob0, ob1,
                sem_in, sem_g, sem_out0, sem_out1):
    nc = 2
    wid = lax.axis_index("s") * nc + lax.axis_index("c")
    per_w = n_queries // NW
    n_chunks = per_w // CHUNK
    plane = h * w
    sets = ((uv0, idx0, ob0, sem_out0), (uv1, idx1, ob1, sem_out1))

    def q0_of(g):
        return wid * per_w + g * CHUNK

    def in_start(g, p):
        uv = sets[p][0]
        pltpu.async_copy(uv_hbm.at[pl.ds(2 * q0_of(g), 2 * CHUNK)], uv, sem_in)

    def in_wait(p):
        uv = sets[p][0]
        pltpu.make_async_copy(uv_hbm.at[pl.ds(0, 2 * CHUNK)], uv, sem_in).wait()

    def comp(p):
        uv, idx = sets[p][0], sets[p][1]

        def body(j, c):
            boff = (j >> 3) * (2 * BLK) + (j & 7) * L
            u = uv[pl.ds(boff, L)]
            v = uv[pl.ds(boff + BLK, L)]
            iu = _nearest(u, float(w - 1))
            iv = _nearest(v, float(h - 1))
            addr = (((iu >> 3) << 15) | ((iv >> 7) << 10)
                    | ((iu & 7) << 7) | (iv & 127))
            off = j * L
            idx[0, pl.ds(off, L)] = addr
            idx[1, pl.ds(off, L)] = addr + plane
            idx[2, pl.ds(off, L)] = addr + 2 * plane
            return c

        lax.fori_loop(0, CHUNK // L, body, 0)

    def gather_start(p):
        idx, ob = sets[p][1], sets[p][2]

        def fire(b, c):
            for ch in range(3):
                pltpu.async_copy(
                    tex_hbm.at[idx.at[ch, pl.ds(b * BLK, BLK)]],
                    ob.at[pl.ds(b * 4 * BLK + ch * BLK, BLK)], sem_g)
            return c

        lax.fori_loop(0, CHUNK // BLK, fire, 0)

    def gather_wait(p):
        ob = sets[p][2]
        pltpu.make_async_copy(tex_hbm.at[pl.ds(0, 3 * CHUNK)],
                              ob.at[pl.ds(0, 3 * CHUNK)], sem_g).wait()

    def out_start(g, p):
        ob, sem = sets[p][2], sets[p][3]
        pltpu.async_copy(ob, out_hbm.at[pl.ds(4 * q0_of(g), 4 * CHUNK)], sem)

    def out_wait(p):
        ob, sem = sets[p][2], sets[p][3]
        pltpu.make_async_copy(ob, out_hbm.at[pl.ds(0, 4 * CHUNK)], sem).wait()

    in_start(0, 0)
    in_wait(0)
    in_start(1, 1)
    comp(0)
    gather_start(0)

    in_start(2, 0)
    in_wait(1)
    comp(1)
    gather_wait(0)
    out_start(0, 0)
    gather_start(1)

    def steady(t, carry):
        for p in (0, 1):
            gk = 2 * t + p
            in_start(jnp.minimum(gk + 1, n_chunks - 1), 1 - p)
            in_wait(p)
            comp(p)
            gather_wait(1 - p)
            out_start(gk - 1, 1 - p)
            out_wait(p)
            gather_start(p)
        return carry

    lax.fori_loop(1, n_chunks // 2, steady, 0)

    gather_wait(1)
    out_start(n_chunks - 1, 1)
    out_wait(0)
    out_wait(1)
    in_wait(0)


def kernel(uvs, texture):
    n = uvs.shape[0]
    h, w, _ = texture.shape
    nblk = n // BLK
    uv_lin = uvs.reshape(nblk, BLK, 2).transpose(0, 2, 1).reshape(2 * n)
    tex_lin = (texture.transpose(2, 0, 1)
               .reshape(3, h // 8, 8, w // 128, 128)
               .transpose(0, 1, 3, 2, 4)
               .reshape(3 * h * w))

    mesh = plsc.VectorSubcoreMesh(core_axis_name="c", subcore_axis_name="s")
    run = functools.partial(
        pl.kernel,
        mesh=mesh,
        out_type=jax.ShapeDtypeStruct((4 * n,), jnp.float32),
        scratch_types=[
            pltpu.VMEM((2 * CHUNK,), jnp.float32),
            pltpu.VMEM((2 * CHUNK,), jnp.float32),
            pltpu.VMEM((3, CHUNK), jnp.int32),
            pltpu.VMEM((3, CHUNK), jnp.int32),
            pltpu.VMEM((4 * CHUNK,), jnp.float32),
            pltpu.VMEM((4 * CHUNK,), jnp.float32),
            pltpu.SemaphoreType.DMA,
            pltpu.SemaphoreType.DMA,
            pltpu.SemaphoreType.DMA,
            pltpu.SemaphoreType.DMA,
        ],
        compiler_params=pltpu.CompilerParams(use_tc_tiling_on_sc=False),
    )(functools.partial(_tex_kernel, n, h, w))
    out_lin = run(uv_lin, tex_lin)
    return (out_lin.reshape(nblk, 4, BLK)[:, :3, :]
            .transpose(0, 2, 1).reshape(n, 3))
